# baseline (speedup 1.0000x reference)
import functools

import jax
import jax.numpy as jnp
from jax import lax
from jax.experimental import pallas as pl
from jax.experimental.pallas import tpu as pltpu
from jax.experimental.pallas import tpu_sc as plsc

N = 10000
E = 320000
D = 128
NUM_LAYER = 5
EPS = 0.5
BN_EPS = 1e-5

NC = 2
NS = 16
NW = NC * NS

N_PAD = 10240
ROWS_W = N_PAD // NW
STRIPE = N_PAD // NS
E_W = E // NW
EK = 125
ECH = E_W // EK
GK = 80
GCH = ROWS_W // GK

_mesh = plsc.VectorSubcoreMesh(
    core_axis_name="c", subcore_axis_name="s", num_cores=NC, num_subcores=NS
)


def _k0_body(emb_ref, idx_ref, h0_ref, idx_v, rows_v, sem):
    c = lax.axis_index("c")
    s = lax.axis_index("s")
    w = c * NS + s

    @pl.loop(0, GCH)
    def _(j):
        pltpu.sync_copy(idx_ref.at[w, j], idx_v)
        pltpu.async_copy(emb_ref.at[idx_v], rows_v, sem).wait()
        pltpu.sync_copy(rows_v, h0_ref.at[pl.ds(w * ROWS_W + j * GK, GK)])


_k0 = pl.kernel(
    _k0_body,
    out_type=jax.ShapeDtypeStruct((N_PAD, D), jnp.float32),
    mesh=_mesh,
    scratch_types=[
        pltpu.VMEM((GK,), jnp.int32),
        pltpu.VMEM((GK, D), jnp.float32),
        pltpu.SemaphoreType.DMA,
    ],
)


def _ka_body(h_ref, sd_ref, out_ref, sd_a, sd_b, rows0, rows1, agg_sh,
             gsem, isem, ssem):
    c = lax.axis_index("c")
    s = lax.axis_index("s")
    w = c * NS + s

    zero16 = jnp.zeros((16,), jnp.float32)

    @pl.loop(0, 80)
    def _(i):
        for j in range(D // 16):
            rows0[i, pl.ds(j * 16, 16)] = zero16

    @pl.loop(0, STRIPE // 80)
    def _(k):
        pltpu.sync_copy(rows0.at[pl.ds(0, 80)],
                        agg_sh.at[pl.ds(s * STRIPE + k * 80, 80)])

    plsc.subcore_barrier()

    pltpu.sync_copy(sd_ref.at[w, 0], sd_a)

    def _quad(sd, pref_desc, pref_late):
        g0 = pltpu.async_copy(h_ref.at[sd.at[0]], rows0, gsem)
        g1 = pltpu.async_copy(h_ref.at[sd.at[2]], rows1, gsem)
        g0.wait()
        s0 = pltpu.async_copy(rows0, agg_sh.at[sd.at[1]], ssem, add=True)
        g1.wait()
        s1 = pltpu.async_copy(rows1, agg_sh.at[sd.at[3]], ssem, add=True)
        s0.wait()
        g2 = pltpu.async_copy(h_ref.at[sd.at[4]], rows0, gsem)
        s1.wait()
        g3 = pltpu.async_copy(h_ref.at[sd.at[6]], rows1, gsem)
        g2.wait()
        s2 = pltpu.async_copy(rows0, agg_sh.at[sd.at[5]], ssem, add=True)
        if pref_desc is not None:
            pref_desc.wait()
        g3.wait()
        s3 = pltpu.async_copy(rows1, agg_sh.at[sd.at[7]], ssem, add=True)
        s2.wait()
        s3.wait()
        return pref_late()

    NG = ECH // 4

    @pl.loop(0, NG, step=2)
    def _(g):
        pb = pltpu.async_copy(sd_ref.at[w, g + 1], sd_b, isem)
        ga = jnp.minimum(g + 2, NG - 1)
        pa = _quad(sd_a, pb,
                   lambda: pltpu.async_copy(sd_ref.at[w, ga], sd_a, isem))
        _quad(sd_b, pa, lambda: None)

    plsc.subcore_barrier()

    pltpu.sync_copy(agg_sh.at[pl.ds(s * STRIPE, STRIPE)],
                    out_ref.at[c, pl.ds(s * STRIPE, STRIPE)])


_ka = pl.kernel(
    _ka_body,
    out_type=jax.ShapeDtypeStruct((NC, N_PAD, D), jnp.float32),
    mesh=_mesh,
    scratch_types=[
        pltpu.VMEM((8, EK), jnp.int32),
        pltpu.VMEM((8, EK), jnp.int32),
        pltpu.VMEM((EK, D), jnp.float32),
        pltpu.VMEM((EK, D), jnp.float32),
        pltpu.VMEM_SHARED((N_PAD, D), jnp.float32),
        pltpu.SemaphoreType.DMA,
        pltpu.SemaphoreType.DMA,
        pltpu.SemaphoreType.DMA,
    ],
)


def _kbc_body(a0_ref, a1_ref, h_ref, out_ref):
    h2 = a0_ref[...] + a1_ref[...] + EPS * h_ref[...]
    rows = lax.broadcasted_iota(jnp.int32, (N_PAD, 1), 0)
    maskf = jnp.where(rows < N, 1.0, 0.0).astype(jnp.float32)
    hm = h2 * maskf
    inv_n = jnp.float32(1.0 / N)
    mean = jnp.sum(hm, axis=0, keepdims=True) * inv_n
    var = jnp.sum(hm * h2, axis=0, keepdims=True) * inv_n - mean * mean
    rs = lax.rsqrt(var + BN_EPS)
    out_ref[...] = (h2 - mean) * rs


_kbc = pl.pallas_call(
    _kbc_body,
    out_shape=jax.ShapeDtypeStruct((N_PAD, D), jnp.float32),
)


@jax.jit
def kernel(x, edge_index, node_embedding):
    idx0 = x[:, 0].astype(jnp.int32)
    idx0_pad = jnp.concatenate(
        [idx0, jnp.full((N_PAD - N,), 120, jnp.int32)]
    ).reshape(NW, GCH, GK)
    emb_pad = jnp.concatenate(
        [node_embedding.astype(jnp.float32), jnp.zeros((8, D), jnp.float32)]
    )
    src_g = edge_index[0].astype(jnp.int32).reshape(NW, ECH // 4, 4, EK)
    dst_g = edge_index[1].astype(jnp.int32).reshape(NW, ECH // 4, 4, EK)
    sd_r = jnp.stack([src_g, dst_g], axis=3).reshape(NW, ECH // 4, 8, EK)

    h = _k0(emb_pad, idx0_pad)

    def layer(_, h):
        agg = _ka(h, sd_r)
        return _kbc(agg[0], agg[1], h)

    h = lax.fori_loop(0, NUM_LAYER, layer, h)
    return h[:N]


# speedup vs baseline: 1.0397x; 1.0397x over previous
import functools

import jax
import jax.numpy as jnp
from jax import lax
from jax.experimental import pallas as pl
from jax.experimental.pallas import tpu as pltpu
from jax.experimental.pallas import tpu_sc as plsc

N = 10000
E = 320000
D = 128
NUM_LAYER = 5
EPS = 0.5
BN_EPS = 1e-5

NC = 2
NS = 16
NW = NC * NS

N_PAD = 10240
ROWS_W = N_PAD // NW
STRIPE = N_PAD // NS
E_W = E // NW
EK = 125
ECH = E_W // EK
GK = 80
GCH = ROWS_W // GK

_mesh = plsc.VectorSubcoreMesh(
    core_axis_name="c", subcore_axis_name="s", num_cores=NC, num_subcores=NS
)


def _k0_body(emb_ref, idx_ref, h0_ref, idx_v, rows_v, sem):
    c = lax.axis_index("c")
    s = lax.axis_index("s")
    w = c * NS + s

    @pl.loop(0, GCH)
    def _(j):
        pltpu.sync_copy(idx_ref.at[w, j], idx_v)
        pltpu.async_copy(emb_ref.at[idx_v], rows_v, sem).wait()
        pltpu.sync_copy(rows_v, h0_ref.at[pl.ds(w * ROWS_W + j * GK, GK)])


_k0 = pl.kernel(
    _k0_body,
    out_type=jax.ShapeDtypeStruct((N_PAD, D), jnp.float32),
    mesh=_mesh,
    scratch_types=[
        pltpu.VMEM((GK,), jnp.int32),
        pltpu.VMEM((GK, D), jnp.float32),
        pltpu.SemaphoreType.DMA,
    ],
)


def _ka_body(h_ref, sd_ref, out_ref, sd_a, sd_b, rows0, rows1, agg_sh,
             gsem, isem, ssem):
    c = lax.axis_index("c")
    s = lax.axis_index("s")
    w = c * NS + s

    zero16 = jnp.zeros((16,), jnp.float32)

    @pl.loop(0, 80)
    def _(i):
        for j in range(D // 16):
            rows0[i, pl.ds(j * 16, 16)] = zero16

    @pl.loop(0, STRIPE // 80)
    def _(k):
        pltpu.sync_copy(rows0.at[pl.ds(0, 80)],
                        agg_sh.at[pl.ds(s * STRIPE + k * 80, 80)])

    plsc.subcore_barrier()

    pltpu.sync_copy(sd_ref.at[w, 0], sd_a)

    def _quad(sd, pref_desc, pref_late):
        g0 = pltpu.async_copy(h_ref.at[sd.at[0]], rows0, gsem)
        g1 = pltpu.async_copy(h_ref.at[sd.at[2]], rows1, gsem)
        g0.wait()
        s0 = pltpu.async_copy(rows0, agg_sh.at[sd.at[1]], ssem, add=True)
        g1.wait()
        s1 = pltpu.async_copy(rows1, agg_sh.at[sd.at[3]], ssem, add=True)
        s0.wait()
        g2 = pltpu.async_copy(h_ref.at[sd.at[4]], rows0, gsem)
        s1.wait()
        g3 = pltpu.async_copy(h_ref.at[sd.at[6]], rows1, gsem)
        g2.wait()
        s2 = pltpu.async_copy(rows0, agg_sh.at[sd.at[5]], ssem, add=True)
        if pref_desc is not None:
            pref_desc.wait()
        g3.wait()
        s3 = pltpu.async_copy(rows1, agg_sh.at[sd.at[7]], ssem, add=True)
        s2.wait()
        s3.wait()
        return pref_late()

    NG = ECH // 4

    @pl.loop(0, NG, step=2)
    def _(g):
        pb = pltpu.async_copy(sd_ref.at[w, g + 1], sd_b, isem)
        ga = jnp.minimum(g + 2, NG - 1)
        pa = _quad(sd_a, pb,
                   lambda: pltpu.async_copy(sd_ref.at[w, ga], sd_a, isem))
        _quad(sd_b, pa, lambda: None)

    plsc.subcore_barrier()

    pltpu.sync_copy(agg_sh.at[pl.ds(s * STRIPE, STRIPE)],
                    out_ref.at[c, pl.ds(s * STRIPE, STRIPE)])


_ka = pl.kernel(
    _ka_body,
    out_type=jax.ShapeDtypeStruct((NC, N_PAD, D), jnp.float32),
    mesh=_mesh,
    scratch_types=[
        pltpu.VMEM((8, EK), jnp.int32),
        pltpu.VMEM((8, EK), jnp.int32),
        pltpu.VMEM((EK, D), jnp.float32),
        pltpu.VMEM((EK, D), jnp.float32),
        pltpu.VMEM_SHARED((N_PAD, D), jnp.float32),
        pltpu.SemaphoreType.DMA,
        pltpu.SemaphoreType.DMA,
        pltpu.SemaphoreType.DMA,
    ],
)


def _kbc_body(a0_ref, a1_ref, h_ref, out_ref):
    h2 = a0_ref[...] + a1_ref[...] + EPS * h_ref[...]
    rows = lax.broadcasted_iota(jnp.int32, (N_PAD, 1), 0)
    maskf = jnp.where(rows < N, 1.0, 0.0).astype(jnp.float32)
    hm = h2 * maskf
    inv_n = jnp.float32(1.0 / N)
    mean = jnp.sum(hm, axis=0, keepdims=True) * inv_n
    var = jnp.sum(hm * h2, axis=0, keepdims=True) * inv_n - mean * mean
    rs = lax.rsqrt(var + BN_EPS)
    out_ref[...] = (h2 - mean) * rs


_kbc = pl.pallas_call(
    _kbc_body,
    out_shape=jax.ShapeDtypeStruct((N_PAD, D), jnp.float32),
)


@jax.jit
def kernel(x, edge_index, node_embedding):
    idx0 = x[:, 0].astype(jnp.int32)
    idx0_pad = jnp.concatenate(
        [idx0, jnp.full((N_PAD - N,), 120, jnp.int32)]
    ).reshape(NW, GCH, GK)
    emb_pad = jnp.concatenate(
        [node_embedding.astype(jnp.float32), jnp.zeros((8, D), jnp.float32)]
    )
    src_g = edge_index[0].astype(jnp.int32).reshape(NW, ECH // 4, 4, EK)
    dst_g = edge_index[1].astype(jnp.int32).reshape(NW, ECH // 4, 4, EK)
    sd_r = jnp.stack([src_g, dst_g], axis=3).reshape(NW, ECH // 4, 8, EK)

    h = _k0(emb_pad, idx0_pad)

    for _ in range(NUM_LAYER):
        agg = _ka(h, sd_r)
        h = _kbc(agg[0], agg[1], h)
    return h[:N]
